# baseline (speedup 1.0000x reference)
import functools

import jax
import jax.numpy as jnp
from jax import lax
from jax.experimental import pallas as pl
from jax.experimental.pallas import tpu as pltpu
from jax.experimental.pallas import tpu_sc as plsc

NC, NS = 2, 16
NW = NC * NS
CHUNK = 128
ROWS_PER_STEP = 2


@functools.partial(jax.jit, static_argnums=(1, 2, 3))
def _embed_gather(args, b, n, d):
    ids, table = args
    b_per_w = b // NW
    n_steps = b_per_w // ROWS_PER_STEP
    mesh = plsc.VectorSubcoreMesh(
        core_axis_name="c", subcore_axis_name="s",
        num_cores=NC, num_subcores=NS)

    @functools.partial(
        pl.kernel,
        out_type=jax.ShapeDtypeStruct((b, n, d), jnp.float32),
        mesh=mesh,
        scratch_types=[
            pltpu.VMEM((b_per_w, n), jnp.int32),
            pltpu.VMEM((ROWS_PER_STEP, n, d), jnp.float32),
            pltpu.VMEM((ROWS_PER_STEP, n, d), jnp.float32),
            pltpu.SemaphoreType.DMA,
            pltpu.SemaphoreType.DMA,
            pltpu.SemaphoreType.DMA,
            pltpu.SemaphoreType.DMA,
        ],
        compiler_params=pltpu.CompilerParams(use_tc_tiling_on_sc=False),
    )
    def k(ids_hbm, table_hbm, out_hbm, idx_v, rows0, rows1, g0, g1, o0, o1):
        wid = lax.axis_index("s") * NC + lax.axis_index("c")
        base = pl.multiple_of(wid * b_per_w, b_per_w)
        pltpu.sync_copy(ids_hbm.at[pl.ds(base, b_per_w)], idx_v)

        bufs = (rows0, rows1)
        gsems = (g0, g1)
        osems = (o0, o1)

        def gather_step(s, p):
            handles = []
            for r in range(ROWS_PER_STEP):
                row = s * ROWS_PER_STEP + r
                for lo, w in ((0, CHUNK), (CHUNK, n - CHUNK)):
                    handles.append(pltpu.async_copy(
                        table_hbm.at[idx_v.at[row, pl.ds(lo, w)]],
                        bufs[p].at[r, pl.ds(lo, w)],
                        gsems[p]))
            for h in handles:
                h.wait()

        def fire_out(s, p):
            off = base + s * ROWS_PER_STEP
            pltpu.async_copy(
                bufs[p], out_hbm.at[pl.ds(off, ROWS_PER_STEP)], osems[p])

        def wait_out(p):
            pltpu.make_async_copy(
                bufs[p], out_hbm.at[pl.ds(0, ROWS_PER_STEP)], osems[p]).wait()

        gather_step(0, 0)
        fire_out(0, 0)
        gather_step(1, 1)
        fire_out(1, 1)

        def pair_body(q, carry):
            for p in (0, 1):
                s = 2 * q + p
                wait_out(p)
                gather_step(s, p)
                fire_out(s, p)
            return carry

        lax.fori_loop(1, n_steps // 2, pair_body, 0)
        wait_out(0)
        wait_out(1)

    return k(ids, table)


def kernel(resource_ids, table):
    b, n = resource_ids.shape
    d = table.shape[1]
    return _embed_gather((resource_ids, table), b, n, d)
